# baseline (speedup 1.0000x reference)
import jax
import jax.numpy as jnp
from jax.experimental import pallas as pl
from jax.experimental.pallas import tpu as pltpu

B, N, C = 8, 1024, 384
K = 1024
N_ITER = 6
T = B * N
TB = 1024
NBLK = T // TB
PAIR = 4
EPS_EMPTY = 1e-20


def _norm_rows(v):
    n = jnp.sqrt(jnp.sum(v * v, axis=-1, keepdims=True))
    return v / jnp.maximum(n, 1e-12)


def _cam_kernel(x_ref, means_ref, out_ref, m_ref, sums_ref):
    m_ref[...] = _norm_rows(means_ref[...])

    def norm_block(b, carry):
        out_ref[pl.ds(b * TB, TB), :] = _norm_rows(x_ref[pl.ds(b * TB, TB), :])
        return carry

    jax.lax.fori_loop(0, NBLK, norm_block, 0)

    lane_iota = jax.lax.broadcasted_iota(jnp.int32, (1, K), 1)

    def dist_block(b):
        return jax.lax.dot_general(
            out_ref[pl.ds(b * TB, TB), :], m_ref[...], (((1,), (1,)), ((), ())),
            preferred_element_type=jnp.float32)

    def onehot(d):
        maxv = jnp.max(d, axis=1, keepdims=True)
        idx = jnp.min(jnp.where(d == maxv, lane_iota, K), axis=1,
                      keepdims=True)
        return (lane_iota == idx).astype(jnp.float32)

    def refine_iter(_, carry):
        sums_ref[...] = EPS_EMPTY * m_ref[...]

        def pair_body(p, carry2):
            b = p * PAIR
            ds = [dist_block(b + j) for j in range(PAIR)]
            for j in range(PAIR):
                oh = onehot(ds[j])
                sums_ref[...] += jax.lax.dot_general(
                    oh, out_ref[pl.ds((b + j) * TB, TB), :],
                    (((0,), (0,)), ((), ())),
                    preferred_element_type=jnp.float32)
            return carry2

        jax.lax.fori_loop(0, NBLK // PAIR, pair_body, 0)
        m_ref[...] = _norm_rows(sums_ref[...])
        return carry

    jax.lax.fori_loop(0, N_ITER - 1, refine_iter, 0)

    m = m_ref[...]
    m_hi = m.astype(jnp.bfloat16)
    m_lo = (m - m_hi.astype(jnp.float32)).astype(jnp.bfloat16)

    def out_pair(p, carry):
        b = p * PAIR
        ds = [dist_block(b + j) for j in range(PAIR)]
        for j in range(PAIR):
            maxv = jnp.max(ds[j], axis=1, keepdims=True)
            idx = jnp.min(jnp.where(ds[j] == maxv, lane_iota, K), axis=1,
                          keepdims=True)
            ohb = (lane_iota == idx).astype(jnp.bfloat16)
            q = (jax.lax.dot_general(
                    ohb, m_hi, (((1,), (0,)), ((), ())),
                    preferred_element_type=jnp.float32)
                 + jax.lax.dot_general(
                    ohb, m_lo, (((1,), (0,)), ((), ())),
                    preferred_element_type=jnp.float32))
            out_ref[pl.ds((b + j) * TB, TB), :] = q
        return carry

    jax.lax.fori_loop(0, NBLK // PAIR, out_pair, 0)


@jax.jit
def kernel(x, means):
    xf = x.reshape(T, C)
    out = pl.pallas_call(
        _cam_kernel,
        out_shape=jax.ShapeDtypeStruct((T, C), jnp.float32),
        scratch_shapes=[
            pltpu.VMEM((K, C), jnp.float32),
            pltpu.VMEM((K, C), jnp.float32),
        ],
    )(xf, means)
    return out.reshape(B, N, C)


# speedup vs baseline: 1.0712x; 1.0712x over previous
import jax
import jax.numpy as jnp
from jax.experimental import pallas as pl
from jax.experimental.pallas import tpu as pltpu

B, N, C = 8, 1024, 384
K = 1024
N_ITER = 6
T = B * N
TB = 1024
NBLK = T // TB
PAIR = 4
EPS_EMPTY = 1e-20


def _norm_rows(v):
    n = jnp.sqrt(jnp.sum(v * v, axis=-1, keepdims=True))
    return v / jnp.maximum(n, 1e-12)


def _cam_kernel(x_ref, means_ref, out_ref, m_ref, sums_ref):
    m_ref[...] = _norm_rows(means_ref[...])

    def norm_block(b, carry):
        out_ref[pl.ds(b * TB, TB), :] = _norm_rows(x_ref[pl.ds(b * TB, TB), :])
        return carry

    jax.lax.fori_loop(0, NBLK, norm_block, 0)

    lane_iota = jax.lax.broadcasted_iota(jnp.int32, (1, K), 1)

    def dist_block(b):
        return jax.lax.dot_general(
            out_ref[pl.ds(b * TB, TB), :], m_ref[...], (((1,), (1,)), ((), ())),
            preferred_element_type=jnp.float32)

    def onehot(d):
        maxv = jnp.max(d, axis=1, keepdims=True)
        idx = jnp.min(jnp.where(d == maxv, lane_iota, K), axis=1,
                      keepdims=True)
        return (lane_iota == idx).astype(jnp.float32)

    def refine_iter(_, carry):
        sums_ref[...] = EPS_EMPTY * m_ref[...]

        def pair_body(p, carry2):
            b = p * PAIR
            ds = [dist_block(b + j) for j in range(PAIR)]
            for j in range(PAIR):
                oh = onehot(ds[j])
                sums_ref[...] += jax.lax.dot_general(
                    oh, out_ref[pl.ds((b + j) * TB, TB), :],
                    (((0,), (0,)), ((), ())),
                    preferred_element_type=jnp.float32)
            return carry2

        jax.lax.fori_loop(0, NBLK // PAIR, pair_body, 0)
        m_ref[...] = _norm_rows(sums_ref[...])
        return carry

    jax.lax.fori_loop(0, N_ITER - 1, refine_iter, 0)

    def out_pair(p, carry):
        b = p * PAIR
        ds = [dist_block(b + j) for j in range(PAIR)]
        for j in range(PAIR):
            oh = onehot(ds[j])
            q = jax.lax.dot_general(
                oh, m_ref[...], (((1,), (0,)), ((), ())),
                preferred_element_type=jnp.float32)
            out_ref[pl.ds((b + j) * TB, TB), :] = q
        return carry

    jax.lax.fori_loop(0, NBLK // PAIR, out_pair, 0)


@jax.jit
def kernel(x, means):
    xf = x.reshape(T, C)
    out = pl.pallas_call(
        _cam_kernel,
        out_shape=jax.ShapeDtypeStruct((T, C), jnp.float32),
        scratch_shapes=[
            pltpu.VMEM((K, C), jnp.float32),
            pltpu.VMEM((K, C), jnp.float32),
        ],
    )(xf, means)
    return out.reshape(B, N, C)
